# baseline (speedup 1.0000x reference)
import jax
import jax.numpy as jnp
from jax.experimental import pallas as pl
from jax.experimental.pallas import tpu as pltpu

E = 384
H = 6
D = 64
SH = 2
WIN = 0.4
QUANT = 0.01
L = 40
L2 = 2 * L - 1
LD = L2 * 3
NB = 256
CW = 41
CZ = 101
CSUM = CW + CW + CZ
WA = D + CSUM + CSUM
WV = D + CSUM

_STRIP_WS = ((WIN, WIN, 1.0), (WIN, 1.0, WIN), (1.0, WIN, WIN))
_STRIP_C = ((CW, CW, CZ), (CW, CZ, CW), (CZ, CW, CW))


def _nbits(c):
    b = 0
    while (1 << b) <= c - 1:
        b += 1
    return b


def _barrel_left(x, amt, nbits):
    W = x.shape[-1]
    for k in range(nbits):
        bit = ((amt >> k) & 1) == 1
        rolled = pltpu.roll(x, W - (1 << k), axis=x.ndim - 1)
        x = jnp.where(bit, rolled, x)
    return x


def _qd_wid(relb, bat, ws):
    widx = []
    qd = []
    for d in range(3):
        x = relb[:, d:d + 1]
        w = jnp.float32(ws[d])
        widx.append(jnp.floor(x / w).astype(jnp.int32))
        qd.append(jnp.floor(jnp.mod(x, w) / jnp.float32(QUANT))
                  .astype(jnp.int32))
    wid = ((bat * 1024 + widx[0]) * 1024 + widx[1]) * 1024 + widx[2]
    return qd, wid


def _aug_kernel(feats_ref, w_ref, b_ref, qt_ref, kt_ref, rel_ref, bat_ref,
                qaug_ref, kaug_ref, vaug_ref, widc_ref):
    x = feats_ref[...]
    w = w_ref[...]
    qkv = jax.lax.dot_general(x, w, (((1,), (1,)), ((), ())),
                              preferred_element_type=jnp.float32)
    qkv = qkv + b_ref[...]
    relb = rel_ref[...]
    bat = bat_ref[...]
    scale = D ** -0.5
    for s in range(3):
        ws = _STRIP_WS[s]
        Cs = _STRIP_C[s]
        qd, wid = _qd_wid(relb, bat, ws)
        widc_ref[s, 0] = wid
        oh = []
        ohrev = []
        for d in range(3):
            C = Cs[d]
            iota = jax.lax.broadcasted_iota(jnp.int32, (NB, C), 1)
            oh.append((iota == qd[d]).astype(jnp.float32))
            ohrev.append((iota == (C - 1 - qd[d])).astype(jnp.float32))
        qs_, ks_, vs_ = [], [], []
        tqd = [[None] * 3 for _ in range(2)]
        tkd = [[None] * 3 for _ in range(2)]
        for hh in range(2):
            h = 2 * s + hh
            qh = qkv[:, h * D:(h + 1) * D] * scale
            kh = qkv[:, E + h * D:E + (h + 1) * D]
            vh = qkv[:, 2 * E + h * D:2 * E + (h + 1) * D]
            qs_.append(qh)
            ks_.append(kh)
            vs_.append(vh)
            for d in range(3):
                tqd[hh][d] = jax.lax.dot_general(
                    qh, qt_ref[hh, d], (((1,), (1,)), ((), ())),
                    preferred_element_type=jnp.float32)
                tkd[hh][d] = jax.lax.dot_general(
                    kh, kt_ref[hh, d], (((1,), (1,)), ((), ())),
                    preferred_element_type=jnp.float32)
        gS = [[None] * 3 for _ in range(2)]
        gT = [[None] * 3 for _ in range(2)]
        groups = {}
        for d in range(3):
            groups.setdefault(Cs[d], []).append(d)
        for C, ds in groups.items():
            WB = 128 if C == CW else 256
            nb = _nbits(C)
            lpad = C - L
            rpad = WB - lpad - L2
            rows, amts = [], []
            for d in ds:
                for src in (tqd, tkd):
                    for hh in range(2):
                        t = src[hh][d]
                        rows.append(jnp.concatenate(
                            [jnp.broadcast_to(t[:, :1], (NB, lpad)), t,
                             jnp.broadcast_to(t[:, L2 - 1:], (NB, rpad))],
                            axis=1)[None])
                a0 = qd[d][None]
                a1 = ((C - 1) - qd[d])[None]
                amts += [a0, a0, a1, a1]
            y = _barrel_left(jnp.concatenate(rows, axis=0),
                             jnp.concatenate(amts, axis=0), nb)
            for gi, d in enumerate(ds):
                for hh in range(2):
                    gS[hh][d] = y[4 * gi + hh, :, :C]
                    gT[hh][d] = y[4 * gi + 2 + hh, :, :C]
        for hh in range(2):
            h = 2 * s + hh
            qaug_ref[h] = jnp.concatenate(
                [qs_[hh], gS[hh][0], gS[hh][1], gS[hh][2],
                 oh[0], oh[1], oh[2]], axis=1)
            kaug_ref[h] = jnp.concatenate(
                [ks_[hh], ohrev[0], ohrev[1], ohrev[2],
                 gT[hh][0], gT[hh][1], gT[hh][2]], axis=1)
            vaug_ref[h] = jnp.concatenate(
                [vs_[hh], oh[0], oh[1], oh[2]], axis=1)


def _dotT(a, b):
    return jax.lax.dot_general(a, b, (((1,), (1,)), ((), ())),
                               preferred_element_type=jnp.float32)


def _dot(a, b):
    return jax.lax.dot_general(a, b, (((1,), (0,)), ((), ())),
                               preferred_element_type=jnp.float32)


def _split(x):
    hi = x.astype(jnp.bfloat16)
    lo = (x - hi.astype(jnp.float32)).astype(jnp.bfloat16)
    return hi, lo


def _attn_kernel(widc_ref, widr_ref, qa_ref, ka_ref, va_ref, o_ref):
    qa = qa_ref[0]
    ka = ka_ref[0]
    qhi, qlo = _split(qa[:, :D])
    khi, klo = _split(ka[:, :D])
    att = _dotT(qhi, khi) + _dotT(qhi, klo) + _dotT(qlo, khi)
    att += _dotT(qa[:, D:].astype(jnp.bfloat16),
                 ka[:, D:].astype(jnp.bfloat16))
    widq = widc_ref[0, 0]
    widr = widr_ref[0]
    mask = widq == widr
    att = jnp.where(mask, att, -1e30)
    m = jnp.max(att, axis=1, keepdims=True)
    p = jnp.exp(att - m)
    s = jnp.sum(p, axis=1, keepdims=True)
    p = p / s
    va = va_ref[0]
    phi, plo = _split(p)
    vahi = va.astype(jnp.bfloat16)
    vlo = (va[:, :D] - vahi[:, :D].astype(jnp.float32)).astype(jnp.bfloat16)
    o = _dot(phi, vahi) + _dot(plo, vahi)
    o64 = _dot(phi, vlo)
    o_ref[0] = jnp.concatenate([o[:, :D] + o64, o[:, D:]], axis=1)


def _combine_proj_kernel(o_ref, rel_ref, bat_ref, vt_ref, pw_ref, pb_ref,
                         out_ref):
    relb = rel_ref[...]
    bat = bat_ref[...]
    parts = []
    for s in range(3):
        ws = _STRIP_WS[s]
        Cs = _STRIP_C[s]
        qd, _ = _qd_wid(relb, bat, ws)
        offs = [D, D + Cs[0], D + Cs[0] + Cs[1]]
        Wrev = [None] * 3
        groups = {}
        for d in range(3):
            groups.setdefault(Cs[d], []).append(d)
        for C, ds in groups.items():
            WB = 128 if C == CW else 256
            nb = _nbits(C)
            rows, amts = [], []
            for d in ds:
                A2 = o_ref[2 * s:2 * s + 2, :, offs[d]:offs[d] + C]
                rows.append(jnp.concatenate(
                    [jnp.zeros((2, NB, L - 1), jnp.float32), A2,
                     jnp.zeros((2, NB, WB - (L - 1) - C), jnp.float32)],
                    axis=2))
                amts.append(jnp.broadcast_to(qd[d][None], (2, NB, 1)))
            y = _barrel_left(jnp.concatenate(rows, axis=0),
                             jnp.concatenate(amts, axis=0), nb)
            for gi, d in enumerate(ds):
                A2 = o_ref[2 * s:2 * s + 2, :, offs[d]:offs[d] + C]
                Wint = y[2 * gi:2 * gi + 2, :, :L2]
                amt = qd[d][None]
                iota = jax.lax.broadcasted_iota(jnp.int32, (2, NB, C), 2)
                sfx = jnp.sum(jnp.where(iota >= amt + (L - 1), A2, 0.0),
                              axis=2, keepdims=True)
                pfx = jnp.sum(jnp.where(iota <= amt - (L - 1), A2, 0.0),
                              axis=2, keepdims=True)
                Wrev[d] = jnp.concatenate(
                    [pfx, Wint[:, :, 1:L2 - 1], sfx], axis=2)
        Wcat = jnp.concatenate(Wrev, axis=2)
        for hh in range(2):
            h = 2 * s + hh
            o2 = jax.lax.dot_general(
                Wcat[hh], vt_ref[h], (((1,), (0,)), ((), ())),
                preferred_element_type=jnp.float32)
            parts.append(o_ref[h, :, :D] + o2)
    y = jnp.concatenate(parts, axis=1)
    out = jax.lax.dot_general(y, pw_ref[...], (((1,), (1,)), ((), ())),
                              preferred_element_type=jnp.float32)
    out_ref[...] = out + pb_ref[...]


def kernel(feats, xyz, batch, qkv_w, qkv_b, proj_w, proj_b,
           q_table, k_table, v_table):
    N = feats.shape[0]
    nblk = N // NB
    f32 = jnp.float32

    mn = xyz.min(axis=0)
    rel = xyz - mn
    bat_col = batch.astype(jnp.int32)[:, None]
    qt3 = q_table.transpose(2, 1, 0, 3)
    kt3 = k_table.transpose(2, 1, 0, 3)

    qaug, kaug, vaug, widc = pl.pallas_call(
        _aug_kernel,
        grid=(nblk,),
        in_specs=[
            pl.BlockSpec((NB, E), lambda i: (i, 0)),
            pl.BlockSpec((3 * E, E), lambda i: (0, 0)),
            pl.BlockSpec((1, 3 * E), lambda i: (0, 0)),
            pl.BlockSpec((SH, 3, L2, D), lambda i: (0, 0, 0, 0)),
            pl.BlockSpec((SH, 3, L2, D), lambda i: (0, 0, 0, 0)),
            pl.BlockSpec((NB, 3), lambda i: (i, 0)),
            pl.BlockSpec((NB, 1), lambda i: (i, 0)),
        ],
        out_specs=[
            pl.BlockSpec((H, NB, WA), lambda i: (0, i, 0)),
            pl.BlockSpec((H, NB, WA), lambda i: (0, i, 0)),
            pl.BlockSpec((H, NB, WV), lambda i: (0, i, 0)),
            pl.BlockSpec((3, 1, NB, 1), lambda i: (0, i, 0, 0)),
        ],
        out_shape=[
            jax.ShapeDtypeStruct((H, N, WA), f32),
            jax.ShapeDtypeStruct((H, N, WA), f32),
            jax.ShapeDtypeStruct((H, N, WV), f32),
            jax.ShapeDtypeStruct((3, nblk, NB, 1), jnp.int32),
        ],
    )(feats, qkv_w, qkv_b[None], qt3, kt3, rel, bat_col)

    widr_l = []
    batch_i = batch.astype(jnp.int32)
    for s in range(3):
        ws = jnp.array(_STRIP_WS[s], dtype=f32)
        w_idx = jnp.floor(rel / ws).astype(jnp.int32)
        wid = ((batch_i * 1024 + w_idx[:, 0]) * 1024 + w_idx[:, 1]) * 1024 \
            + w_idx[:, 2]
        widr_l.append(wid.reshape(1, N))
    widr = jnp.stack(widr_l, axis=0)

    O = pl.pallas_call(
        _attn_kernel,
        grid=(H, nblk),
        in_specs=[
            pl.BlockSpec((1, 1, NB, 1), lambda h, i: (h // 2, i, 0, 0)),
            pl.BlockSpec((1, 1, N), lambda h, i: (h // 2, 0, 0)),
            pl.BlockSpec((1, NB, WA), lambda h, i: (h, i, 0)),
            pl.BlockSpec((1, N, WA), lambda h, i: (h, 0, 0)),
            pl.BlockSpec((1, N, WV), lambda h, i: (h, 0, 0)),
        ],
        out_specs=pl.BlockSpec((1, NB, WV), lambda h, i: (h, i, 0)),
        out_shape=jax.ShapeDtypeStruct((H, N, WV), f32),
    )(widc, widr, qaug, kaug, vaug)

    vt_rev = v_table[::-1].transpose(1, 0, 2, 3).reshape(LD, SH, D)
    vt6 = jnp.stack([vt_rev[:, h % SH, :] for h in range(H)], axis=0)

    out = pl.pallas_call(
        _combine_proj_kernel,
        grid=(nblk,),
        in_specs=[
            pl.BlockSpec((H, NB, WV), lambda i: (0, i, 0)),
            pl.BlockSpec((NB, 3), lambda i: (i, 0)),
            pl.BlockSpec((NB, 1), lambda i: (i, 0)),
            pl.BlockSpec((H, LD, D), lambda i: (0, 0, 0)),
            pl.BlockSpec((E, E), lambda i: (0, 0)),
            pl.BlockSpec((1, E), lambda i: (0, 0)),
        ],
        out_specs=pl.BlockSpec((NB, E), lambda i: (i, 0)),
        out_shape=jax.ShapeDtypeStruct((N, E), f32),
    )(O, rel, bat_col, vt6, proj_w, proj_b[None])
    return out


# speedup vs baseline: 1.0516x; 1.0516x over previous
import jax
import jax.numpy as jnp
from jax.experimental import pallas as pl
from jax.experimental.pallas import tpu as pltpu

E = 384
H = 6
D = 64
SH = 2
WIN = 0.4
QUANT = 0.01
L = 40
L2 = 2 * L - 1
LD = L2 * 3
NBA = 128
NBB = 256
CW = 41
CZ = 101
CSUM = CW + CW + CZ
WA = D + CSUM + CSUM
WV = D + CSUM

_STRIP_WS = ((WIN, WIN, 1.0), (WIN, 1.0, WIN), (1.0, WIN, WIN))
_STRIP_C = ((CW, CW, CZ), (CW, CZ, CW), (CZ, CW, CW))


def _nbits(c):
    b = 0
    while (1 << b) <= c - 1:
        b += 1
    return b


def _barrel_left(x, amt, nbits):
    W = x.shape[-1]
    for k in range(nbits):
        bit = ((amt >> k) & 1) == 1
        rolled = pltpu.roll(x, W - (1 << k), axis=x.ndim - 1)
        x = jnp.where(bit, rolled, x)
    return x


def _qd_wid(relb, bat, ws):
    widx = []
    qd = []
    for d in range(3):
        x = relb[:, d:d + 1]
        w = jnp.float32(ws[d])
        widx.append(jnp.floor(x / w).astype(jnp.int32))
        qd.append(jnp.floor(jnp.mod(x, w) / jnp.float32(QUANT))
                  .astype(jnp.int32))
    wid = ((bat * 1024 + widx[0]) * 1024 + widx[1]) * 1024 + widx[2]
    return qd, wid


def _aug_kernel(feats_ref, w_ref, b_ref, qt_ref, kt_ref, rel_ref, bat_ref,
                qaug_ref, kaug_ref, vaug_ref, widc_ref):
    x = feats_ref[...]
    w = w_ref[...]
    qkv = jax.lax.dot_general(x, w, (((1,), (1,)), ((), ())),
                              preferred_element_type=jnp.float32)
    qkv = qkv + b_ref[...]
    relb = rel_ref[...]
    bat = bat_ref[...]
    scale = D ** -0.5
    for s in range(3):
        ws = _STRIP_WS[s]
        Cs = _STRIP_C[s]
        qd, wid = _qd_wid(relb, bat, ws)
        widc_ref[s] = wid
        oh = []
        ohrev = []
        for d in range(3):
            C = Cs[d]
            iota = jax.lax.broadcasted_iota(jnp.int32, (NBA, C), 1)
            oh.append((iota == qd[d]).astype(jnp.float32))
            ohrev.append((iota == (C - 1 - qd[d])).astype(jnp.float32))
        qs_, ks_, vs_ = [], [], []
        tqd = [[None] * 3 for _ in range(2)]
        tkd = [[None] * 3 for _ in range(2)]
        for hh in range(2):
            h = 2 * s + hh
            qh = qkv[:, h * D:(h + 1) * D] * scale
            kh = qkv[:, E + h * D:E + (h + 1) * D]
            vh = qkv[:, 2 * E + h * D:2 * E + (h + 1) * D]
            qs_.append(qh)
            ks_.append(kh)
            vs_.append(vh)
            for d in range(3):
                tqd[hh][d] = jax.lax.dot_general(
                    qh, qt_ref[hh, d], (((1,), (1,)), ((), ())),
                    preferred_element_type=jnp.float32)
                tkd[hh][d] = jax.lax.dot_general(
                    kh, kt_ref[hh, d], (((1,), (1,)), ((), ())),
                    preferred_element_type=jnp.float32)
        gS = [[None] * 3 for _ in range(2)]
        gT = [[None] * 3 for _ in range(2)]
        groups = {}
        for d in range(3):
            groups.setdefault(Cs[d], []).append(d)
        for C, ds in groups.items():
            WB = 128 if C == CW else 256
            nb = _nbits(C)
            lpad = C - L
            rpad = WB - lpad - L2
            rows, amts = [], []
            for d in ds:
                for src in (tqd, tkd):
                    for hh in range(2):
                        t = src[hh][d]
                        rows.append(jnp.concatenate(
                            [jnp.broadcast_to(t[:, :1], (NBA, lpad)), t,
                             jnp.broadcast_to(t[:, L2 - 1:], (NBA, rpad))],
                            axis=1)[None])
                a0 = qd[d][None]
                a1 = ((C - 1) - qd[d])[None]
                amts += [a0, a0, a1, a1]
            y = _barrel_left(jnp.concatenate(rows, axis=0),
                             jnp.concatenate(amts, axis=0), nb)
            for gi, d in enumerate(ds):
                for hh in range(2):
                    gS[hh][d] = y[4 * gi + hh, :, :C]
                    gT[hh][d] = y[4 * gi + 2 + hh, :, :C]
        for hh in range(2):
            h = 2 * s + hh
            qaug_ref[h] = jnp.concatenate(
                [qs_[hh], gS[hh][0], gS[hh][1], gS[hh][2],
                 oh[0], oh[1], oh[2]], axis=1)
            kaug_ref[h] = jnp.concatenate(
                [ks_[hh], ohrev[0], ohrev[1], ohrev[2],
                 gT[hh][0], gT[hh][1], gT[hh][2]], axis=1)
            vaug_ref[h] = jnp.concatenate(
                [vs_[hh], oh[0], oh[1], oh[2]], axis=1)


def _dotT(a, b):
    return jax.lax.dot_general(a, b, (((1,), (1,)), ((), ())),
                               preferred_element_type=jnp.float32)


def _dot(a, b):
    return jax.lax.dot_general(a, b, (((1,), (0,)), ((), ())),
                               preferred_element_type=jnp.float32)


def _split(x):
    hi = x.astype(jnp.bfloat16)
    lo = (x - hi.astype(jnp.float32)).astype(jnp.bfloat16)
    return hi, lo


def _attn_kernel(widc_ref, widr_ref, qa_ref, ka_ref, va_ref, o_ref):
    qa = qa_ref[0]
    ka = ka_ref[0]
    att = _dotT(qa, ka)
    widq = widc_ref[0]
    widr = widr_ref[0]
    mask = widq == widr
    att = jnp.where(mask, att, -1e30)
    m = jnp.max(att, axis=1, keepdims=True)
    p = jnp.exp(att - m)
    s = jnp.sum(p, axis=1, keepdims=True)
    p = p / s
    o_ref[0] = _dot(p, va_ref[0])


def _combine_proj_kernel(o_ref, rel_ref, bat_ref, vt_ref, pw_ref, pb_ref,
                         out_ref):
    relb = rel_ref[...]
    bat = bat_ref[...]
    parts = []
    for s in range(3):
        ws = _STRIP_WS[s]
        Cs = _STRIP_C[s]
        qd, _ = _qd_wid(relb, bat, ws)
        offs = [D, D + Cs[0], D + Cs[0] + Cs[1]]
        Wrev = [None] * 3
        groups = {}
        for d in range(3):
            groups.setdefault(Cs[d], []).append(d)
        for C, ds in groups.items():
            WB = 128 if C == CW else 256
            nb = _nbits(C)
            rows, amts = [], []
            for d in ds:
                A2 = o_ref[2 * s:2 * s + 2, :, offs[d]:offs[d] + C]
                rows.append(jnp.concatenate(
                    [jnp.zeros((2, NBA, L - 1), jnp.float32), A2,
                     jnp.zeros((2, NBA, WB - (L - 1) - C), jnp.float32)],
                    axis=2))
                amts.append(jnp.broadcast_to(qd[d][None], (2, NBA, 1)))
            y = _barrel_left(jnp.concatenate(rows, axis=0),
                             jnp.concatenate(amts, axis=0), nb)
            for gi, d in enumerate(ds):
                A2 = o_ref[2 * s:2 * s + 2, :, offs[d]:offs[d] + C]
                Wint = y[2 * gi:2 * gi + 2, :, :L2]
                amt = qd[d][None]
                iota = jax.lax.broadcasted_iota(jnp.int32, (2, NBA, C), 2)
                sfx = jnp.sum(jnp.where(iota >= amt + (L - 1), A2, 0.0),
                              axis=2, keepdims=True)
                pfx = jnp.sum(jnp.where(iota <= amt - (L - 1), A2, 0.0),
                              axis=2, keepdims=True)
                Wrev[d] = jnp.concatenate(
                    [pfx, Wint[:, :, 1:L2 - 1], sfx], axis=2)
        Wcat = jnp.concatenate(Wrev, axis=2)
        for hh in range(2):
            h = 2 * s + hh
            o2 = jax.lax.dot_general(
                Wcat[hh], vt_ref[h], (((1,), (0,)), ((), ())),
                preferred_element_type=jnp.float32)
            parts.append(o_ref[h, :, :D] + o2)
    y = jnp.concatenate(parts, axis=1)
    out = jax.lax.dot_general(y, pw_ref[...], (((1,), (1,)), ((), ())),
                              preferred_element_type=jnp.float32)
    out_ref[...] = out + pb_ref[...]


def kernel(feats, xyz, batch, qkv_w, qkv_b, proj_w, proj_b,
           q_table, k_table, v_table):
    N = feats.shape[0]
    nblka = N // NBA
    nblkb = N // NBB
    f32 = jnp.float32

    mn = xyz.min(axis=0)
    rel = xyz - mn
    bat_col = batch.astype(jnp.int32)[:, None]
    qt3 = q_table.transpose(2, 1, 0, 3)
    kt3 = k_table.transpose(2, 1, 0, 3)

    qaug, kaug, vaug, widc = pl.pallas_call(
        _aug_kernel,
        grid=(nblka,),
        in_specs=[
            pl.BlockSpec((NBA, E), lambda i: (i, 0)),
            pl.BlockSpec((3 * E, E), lambda i: (0, 0)),
            pl.BlockSpec((1, 3 * E), lambda i: (0, 0)),
            pl.BlockSpec((SH, 3, L2, D), lambda i: (0, 0, 0, 0)),
            pl.BlockSpec((SH, 3, L2, D), lambda i: (0, 0, 0, 0)),
            pl.BlockSpec((NBA, 3), lambda i: (i, 0)),
            pl.BlockSpec((NBA, 1), lambda i: (i, 0)),
        ],
        out_specs=[
            pl.BlockSpec((H, NBA, WA), lambda i: (0, i, 0)),
            pl.BlockSpec((H, NBA, WA), lambda i: (0, i, 0)),
            pl.BlockSpec((H, NBA, WV), lambda i: (0, i, 0)),
            pl.BlockSpec((3, NBA, 1), lambda i: (0, i, 0)),
        ],
        out_shape=[
            jax.ShapeDtypeStruct((H, N, WA), f32),
            jax.ShapeDtypeStruct((H, N, WA), f32),
            jax.ShapeDtypeStruct((H, N, WV), f32),
            jax.ShapeDtypeStruct((3, N, 1), jnp.int32),
        ],
    )(feats, qkv_w, qkv_b[None], qt3, kt3, rel, bat_col)

    widr_l = []
    batch_i = batch.astype(jnp.int32)
    for s in range(3):
        ws = jnp.array(_STRIP_WS[s], dtype=f32)
        w_idx = jnp.floor(rel / ws).astype(jnp.int32)
        wid = ((batch_i * 1024 + w_idx[:, 0]) * 1024 + w_idx[:, 1]) * 1024 \
            + w_idx[:, 2]
        widr_l.append(wid.reshape(1, N))
    widr = jnp.stack(widr_l, axis=0)

    O = pl.pallas_call(
        _attn_kernel,
        grid=(H, nblkb),
        in_specs=[
            pl.BlockSpec((1, NBB, 1), lambda h, i: (h // 2, i, 0)),
            pl.BlockSpec((1, 1, N), lambda h, i: (h // 2, 0, 0)),
            pl.BlockSpec((1, NBB, WA), lambda h, i: (h, i, 0)),
            pl.BlockSpec((1, N, WA), lambda h, i: (h, 0, 0)),
            pl.BlockSpec((1, N, WV), lambda h, i: (h, 0, 0)),
        ],
        out_specs=pl.BlockSpec((1, NBB, WV), lambda h, i: (h, i, 0)),
        out_shape=jax.ShapeDtypeStruct((H, N, WV), f32),
    )(widc, widr, qaug, kaug, vaug)

    vt_rev = v_table[::-1].transpose(1, 0, 2, 3).reshape(LD, SH, D)
    vt6 = jnp.stack([vt_rev[:, h % SH, :] for h in range(H)], axis=0)

    out = pl.pallas_call(
        _combine_proj_kernel,
        grid=(nblka,),
        in_specs=[
            pl.BlockSpec((H, NBA, WV), lambda i: (0, i, 0)),
            pl.BlockSpec((NBA, 3), lambda i: (i, 0)),
            pl.BlockSpec((NBA, 1), lambda i: (i, 0)),
            pl.BlockSpec((H, LD, D), lambda i: (0, 0, 0)),
            pl.BlockSpec((E, E), lambda i: (0, 0)),
            pl.BlockSpec((1, E), lambda i: (0, 0)),
        ],
        out_specs=pl.BlockSpec((NBA, E), lambda i: (i, 0)),
        out_shape=jax.ShapeDtypeStruct((N, E), f32),
    )(O, rel, bat_col, vt6, proj_w, proj_b[None])
    return out


# speedup vs baseline: 1.1499x; 1.0935x over previous
import jax
import jax.numpy as jnp
from jax.experimental import pallas as pl
from jax.experimental.pallas import tpu as pltpu

E = 384
H = 6
D = 64
SH = 2
WIN = 0.4
QUANT = 0.01
L = 40
L2 = 2 * L - 1
LD = L2 * 3
NBA = 256
NBB = 256
CW = 41
CZ = 101
CSUM = CW + CW + CZ
WA = D + CSUM + CSUM
WV = D + CSUM

_STRIP_WS = ((WIN, WIN, 1.0), (WIN, 1.0, WIN), (1.0, WIN, WIN))
_STRIP_C = ((CW, CW, CZ), (CW, CZ, CW), (CZ, CW, CW))


def _nbits(c):
    b = 0
    while (1 << b) <= c - 1:
        b += 1
    return b


def _barrel_left(x, amt, nbits):
    W = x.shape[-1]
    for k in range(nbits):
        bit = ((amt >> k) & 1) == 1
        rolled = pltpu.roll(x, W - (1 << k), axis=x.ndim - 1)
        x = jnp.where(bit, rolled, x)
    return x


def _qd_wid(relb, bat, ws):
    widx = []
    qd = []
    for d in range(3):
        x = relb[:, d:d + 1]
        w = jnp.float32(ws[d])
        widx.append(jnp.floor(x / w).astype(jnp.int32))
        qd.append(jnp.floor(jnp.mod(x, w) / jnp.float32(QUANT))
                  .astype(jnp.int32))
    wid = ((bat * 1024 + widx[0]) * 1024 + widx[1]) * 1024 + widx[2]
    return qd, wid


def _aug_kernel(feats_ref, w_ref, b_ref, qt_ref, kt_ref, rel_ref, bat_ref,
                qaug_ref, kaug_ref, vaug_ref, widc_ref):
    x = feats_ref[...]
    w = w_ref[...]
    qkv = jax.lax.dot_general(x, w, (((1,), (1,)), ((), ())),
                              preferred_element_type=jnp.float32)
    qkv = qkv + b_ref[...]
    relb = rel_ref[...]
    bat = bat_ref[...]
    scale = D ** -0.5
    for s in range(3):
        ws = _STRIP_WS[s]
        Cs = _STRIP_C[s]
        qd, wid = _qd_wid(relb, bat, ws)
        widc_ref[s] = wid
        oh = []
        ohrev = []
        for d in range(3):
            C = Cs[d]
            iota = jax.lax.broadcasted_iota(jnp.int32, (NBA, C), 1)
            oh.append((iota == qd[d]).astype(jnp.float32))
            ohrev.append((iota == (C - 1 - qd[d])).astype(jnp.float32))
        qs_, ks_, vs_ = [], [], []
        tqd = [[None] * 3 for _ in range(2)]
        tkd = [[None] * 3 for _ in range(2)]
        for hh in range(2):
            h = 2 * s + hh
            qh = qkv[:, h * D:(h + 1) * D] * scale
            kh = qkv[:, E + h * D:E + (h + 1) * D]
            vh = qkv[:, 2 * E + h * D:2 * E + (h + 1) * D]
            qs_.append(qh)
            ks_.append(kh)
            vs_.append(vh)
            for d in range(3):
                tqd[hh][d] = jax.lax.dot_general(
                    qh, qt_ref[hh, d], (((1,), (1,)), ((), ())),
                    preferred_element_type=jnp.float32)
                tkd[hh][d] = jax.lax.dot_general(
                    kh, kt_ref[hh, d], (((1,), (1,)), ((), ())),
                    preferred_element_type=jnp.float32)
        gS = [[None] * 3 for _ in range(2)]
        gT = [[None] * 3 for _ in range(2)]
        groups = {}
        for d in range(3):
            groups.setdefault(Cs[d], []).append(d)
        for C, ds in groups.items():
            WB = 128 if C == CW else 256
            nb = _nbits(C)
            lpad = C - L
            rpad = WB - lpad - L2
            rows, amts = [], []
            for d in ds:
                for src in (tqd, tkd):
                    for hh in range(2):
                        t = src[hh][d]
                        rows.append(jnp.concatenate(
                            [jnp.broadcast_to(t[:, :1], (NBA, lpad)), t,
                             jnp.broadcast_to(t[:, L2 - 1:], (NBA, rpad))],
                            axis=1)[None])
                a0 = qd[d][None]
                a1 = ((C - 1) - qd[d])[None]
                amts += [a0, a0, a1, a1]
            y = _barrel_left(jnp.concatenate(rows, axis=0),
                             jnp.concatenate(amts, axis=0), nb)
            for gi, d in enumerate(ds):
                for hh in range(2):
                    gS[hh][d] = y[4 * gi + hh, :, :C]
                    gT[hh][d] = y[4 * gi + 2 + hh, :, :C]
        for hh in range(2):
            h = 2 * s + hh
            qaug_ref[h] = jnp.concatenate(
                [qs_[hh], gS[hh][0], gS[hh][1], gS[hh][2],
                 oh[0], oh[1], oh[2]], axis=1)
            kaug_ref[h] = jnp.concatenate(
                [ks_[hh], ohrev[0], ohrev[1], ohrev[2],
                 gT[hh][0], gT[hh][1], gT[hh][2]], axis=1)
            vaug_ref[h] = jnp.concatenate(
                [vs_[hh], oh[0], oh[1], oh[2]], axis=1)


def _dotT(a, b):
    return jax.lax.dot_general(a, b, (((1,), (1,)), ((), ())),
                               preferred_element_type=jnp.float32)


def _dot(a, b):
    return jax.lax.dot_general(a, b, (((1,), (0,)), ((), ())),
                               preferred_element_type=jnp.float32)


def _split(x):
    hi = x.astype(jnp.bfloat16)
    lo = (x - hi.astype(jnp.float32)).astype(jnp.bfloat16)
    return hi, lo


def _attn_kernel(widc_ref, widr_ref, qa_ref, ka_ref, va_ref, o_ref):
    qa = qa_ref[0]
    ka = ka_ref[0]
    att = _dotT(qa, ka)
    widq = widc_ref[0]
    widr = widr_ref[0]
    mask = widq == widr
    att = jnp.where(mask, att, -1e30)
    m = jnp.max(att, axis=1, keepdims=True)
    p = jnp.exp(att - m)
    s = jnp.sum(p, axis=1, keepdims=True)
    p = p / s
    o_ref[0] = _dot(p, va_ref[0])


def _combine_proj_kernel(o_ref, rel_ref, bat_ref, vt_ref, pw_ref, pb_ref,
                         out_ref):
    relb = rel_ref[...]
    bat = bat_ref[...]
    parts = []
    for s in range(3):
        ws = _STRIP_WS[s]
        Cs = _STRIP_C[s]
        qd, _ = _qd_wid(relb, bat, ws)
        offs = [D, D + Cs[0], D + Cs[0] + Cs[1]]
        Wrev = [None] * 3
        groups = {}
        for d in range(3):
            groups.setdefault(Cs[d], []).append(d)
        for C, ds in groups.items():
            WB = 128 if C == CW else 256
            nb = _nbits(C)
            rows, amts = [], []
            for d in ds:
                A2 = o_ref[2 * s:2 * s + 2, :, offs[d]:offs[d] + C]
                rows.append(jnp.concatenate(
                    [jnp.zeros((2, NBA, L - 1), jnp.float32), A2,
                     jnp.zeros((2, NBA, WB - (L - 1) - C), jnp.float32)],
                    axis=2))
                amts.append(jnp.broadcast_to(qd[d][None], (2, NBA, 1)))
            y = _barrel_left(jnp.concatenate(rows, axis=0),
                             jnp.concatenate(amts, axis=0), nb)
            for gi, d in enumerate(ds):
                A2 = o_ref[2 * s:2 * s + 2, :, offs[d]:offs[d] + C]
                Wint = y[2 * gi:2 * gi + 2, :, :L2]
                amt = qd[d][None]
                iota = jax.lax.broadcasted_iota(jnp.int32, (2, NBA, C), 2)
                sfx = jnp.sum(jnp.where(iota >= amt + (L - 1), A2, 0.0),
                              axis=2, keepdims=True)
                pfx = jnp.sum(jnp.where(iota <= amt - (L - 1), A2, 0.0),
                              axis=2, keepdims=True)
                Wrev[d] = jnp.concatenate(
                    [pfx, Wint[:, :, 1:L2 - 1], sfx], axis=2)
        Wcat = jnp.concatenate(Wrev, axis=2)
        for hh in range(2):
            h = 2 * s + hh
            o2 = jax.lax.dot_general(
                Wcat[hh], vt_ref[h], (((1,), (0,)), ((), ())),
                preferred_element_type=jnp.float32)
            parts.append(o_ref[h, :, :D] + o2)
    y = jnp.concatenate(parts, axis=1)
    out = jax.lax.dot_general(y, pw_ref[...], (((1,), (1,)), ((), ())),
                              preferred_element_type=jnp.float32)
    out_ref[...] = out + pb_ref[...]


def kernel(feats, xyz, batch, qkv_w, qkv_b, proj_w, proj_b,
           q_table, k_table, v_table):
    N = feats.shape[0]
    nblka = N // NBA
    nblkb = N // NBB
    f32 = jnp.float32

    mn = xyz.min(axis=0)
    rel = xyz - mn
    bat_col = batch.astype(jnp.int32)[:, None]
    qt3 = q_table.transpose(2, 1, 0, 3)
    kt3 = k_table.transpose(2, 1, 0, 3)

    qaug, kaug, vaug, widc = pl.pallas_call(
        _aug_kernel,
        grid=(nblka,),
        in_specs=[
            pl.BlockSpec((NBA, E), lambda i: (i, 0)),
            pl.BlockSpec((3 * E, E), lambda i: (0, 0)),
            pl.BlockSpec((1, 3 * E), lambda i: (0, 0)),
            pl.BlockSpec((SH, 3, L2, D), lambda i: (0, 0, 0, 0)),
            pl.BlockSpec((SH, 3, L2, D), lambda i: (0, 0, 0, 0)),
            pl.BlockSpec((NBA, 3), lambda i: (i, 0)),
            pl.BlockSpec((NBA, 1), lambda i: (i, 0)),
        ],
        out_specs=[
            pl.BlockSpec((H, NBA, WA), lambda i: (0, i, 0)),
            pl.BlockSpec((H, NBA, WA), lambda i: (0, i, 0)),
            pl.BlockSpec((H, NBA, WV), lambda i: (0, i, 0)),
            pl.BlockSpec((3, NBA, 1), lambda i: (0, i, 0)),
        ],
        out_shape=[
            jax.ShapeDtypeStruct((H, N, WA), f32),
            jax.ShapeDtypeStruct((H, N, WA), f32),
            jax.ShapeDtypeStruct((H, N, WV), f32),
            jax.ShapeDtypeStruct((3, N, 1), jnp.int32),
        ],
    )(feats, qkv_w, qkv_b[None], qt3, kt3, rel, bat_col)

    widr_l = []
    batch_i = batch.astype(jnp.int32)
    for s in range(3):
        ws = jnp.array(_STRIP_WS[s], dtype=f32)
        w_idx = jnp.floor(rel / ws).astype(jnp.int32)
        wid = ((batch_i * 1024 + w_idx[:, 0]) * 1024 + w_idx[:, 1]) * 1024 \
            + w_idx[:, 2]
        widr_l.append(wid.reshape(1, N))
    widr = jnp.stack(widr_l, axis=0)

    O = pl.pallas_call(
        _attn_kernel,
        grid=(H, nblkb),
        in_specs=[
            pl.BlockSpec((1, NBB, 1), lambda h, i: (h // 2, i, 0)),
            pl.BlockSpec((1, 1, N), lambda h, i: (h // 2, 0, 0)),
            pl.BlockSpec((1, NBB, WA), lambda h, i: (h, i, 0)),
            pl.BlockSpec((1, N, WA), lambda h, i: (h, 0, 0)),
            pl.BlockSpec((1, N, WV), lambda h, i: (h, 0, 0)),
        ],
        out_specs=pl.BlockSpec((1, NBB, WV), lambda h, i: (h, i, 0)),
        out_shape=jax.ShapeDtypeStruct((H, N, WV), f32),
    )(widc, widr, qaug, kaug, vaug)

    vt_rev = v_table[::-1].transpose(1, 0, 2, 3).reshape(LD, SH, D)
    vt6 = jnp.stack([vt_rev[:, h % SH, :] for h in range(H)], axis=0)

    out = pl.pallas_call(
        _combine_proj_kernel,
        grid=(nblka,),
        in_specs=[
            pl.BlockSpec((H, NBA, WV), lambda i: (0, i, 0)),
            pl.BlockSpec((NBA, 3), lambda i: (i, 0)),
            pl.BlockSpec((NBA, 1), lambda i: (i, 0)),
            pl.BlockSpec((H, LD, D), lambda i: (0, 0, 0)),
            pl.BlockSpec((E, E), lambda i: (0, 0)),
            pl.BlockSpec((1, E), lambda i: (0, 0)),
        ],
        out_specs=pl.BlockSpec((NBA, E), lambda i: (i, 0)),
        out_shape=jax.ShapeDtypeStruct((N, E), f32),
    )(O, rel, bat_col, vt6, proj_w, proj_b[None])
    return out
